# Initial kernel scaffold; baseline (speedup 1.0000x reference)
import jax
import jax.numpy as jnp
from jax.experimental import pallas as pl


def kernel(x, edge_index, edge_attr, weight):
    raise NotImplementedError("write your pallas kernel here")


# speedup vs baseline: 2.3016x; 2.3016x over previous
import functools

import jax
import jax.numpy as jnp
from jax import lax
from jax.experimental import pallas as pl
from jax.experimental.pallas import tpu as pltpu
from jax.experimental.pallas import tpu_sc as plsc

N_NODES = 10000
N_EDGES = 160000
F = 128
NBASIS = 4
YW = NBASIS * F

CHUNK = 64
NCHUNKS = N_EDGES // CHUNK
NC, NS = 2, 16
NW = NC * NS
NPAD = 10240
ROWS_PER_TILE = NPAD // NS
ZROWS = 32

_MM_BLOCK = 1000


def _mm_body(x_ref, w_ref, y_ref):
    y_ref[...] = jnp.dot(x_ref[...], w_ref[...],
                         preferred_element_type=jnp.float32)


def _add_body(p_ref, o_ref):
    o_ref[...] = p_ref[0] + p_ref[1]


def _sc_body(y_hbm, pk_hbm, part_hbm, pk_v, rows_v, msg_v, zb_v, acc_sh, sem):
    c = lax.axis_index("c")
    s = lax.axis_index("s")
    w = s * NC + c

    @pl.loop(0, ZROWS)
    def _zero_rows(r):
        for fb in range(F // 16):
            zb_v[r, pl.ds(fb * 16, 16)] = jnp.zeros((16,), jnp.float32)

    for t in range(ROWS_PER_TILE // ZROWS):
        pltpu.sync_copy(zb_v, acc_sh.at[pl.ds(s * ROWS_PER_TILE + t * ZROWS,
                                              ZROWS)])
    plsc.subcore_barrier()

    n_chunks = (NCHUNKS - w + NW - 1) // NW

    @pl.loop(0, n_chunks)
    def _chunk(n):
        chunk = w + n * NW
        pltpu.sync_copy(pk_hbm.at[chunk], pk_v)
        pltpu.async_copy(y_hbm.at[pk_v.at[3]], rows_v, sem).wait()

        @pl.loop(0, CHUNK // 16)
        def _grp(g):
            pvec = lax.bitcast_convert_type(pk_v[0, pl.ds(g * 16, 16)], jnp.float32)
            qvec = lax.bitcast_convert_type(pk_v[1, pl.ds(g * 16, 16)], jnp.float32)
            one = jnp.float32(1.0)
            half = jnp.float32(0.5)
            zero = jnp.float32(0.0)
            u0 = jnp.maximum(zero, one - half * jnp.abs(pvec + one))
            u1 = jnp.maximum(zero, one - half * jnp.abs(pvec - one))
            v0 = jnp.maximum(zero, one - half * jnp.abs(qvec + one))
            v1 = jnp.maximum(zero, one - half * jnp.abs(qvec - one))
            c00 = u0 * v0
            c01 = u0 * v1
            c10 = u1 * v0
            c11 = u1 * v1
            for k in range(16):
                e = g * 16 + k
                b0 = jnp.full((16,), c00[k])
                b1 = jnp.full((16,), c01[k])
                b2 = jnp.full((16,), c10[k])
                b3 = jnp.full((16,), c11[k])
                for fb in range(F // 16):
                    o = fb * 16
                    acc = rows_v[e, pl.ds(o, 16)] * b0
                    acc = acc + rows_v[e, pl.ds(F + o, 16)] * b1
                    acc = acc + rows_v[e, pl.ds(2 * F + o, 16)] * b2
                    acc = acc + rows_v[e, pl.ds(3 * F + o, 16)] * b3
                    msg_v[e, pl.ds(o, 16)] = acc

        pltpu.sync_copy(msg_v, acc_sh.at[pk_v.at[2]], add=True)

    plsc.subcore_barrier()
    pltpu.sync_copy(acc_sh.at[pl.ds(s * ROWS_PER_TILE, ROWS_PER_TILE)],
                    part_hbm.at[c, pl.ds(s * ROWS_PER_TILE, ROWS_PER_TILE)])


_sc_call = pl.kernel(
    _sc_body,
    out_type=jax.ShapeDtypeStruct((NC, NPAD, F), jnp.float32),
    mesh=plsc.VectorSubcoreMesh(core_axis_name="c", subcore_axis_name="s"),
    scratch_types=[
        pltpu.VMEM((4, CHUNK), jnp.int32),
        pltpu.VMEM((CHUNK, YW), jnp.float32),
        pltpu.VMEM((CHUNK, F), jnp.float32),
        pltpu.VMEM((ZROWS, F), jnp.float32),
        pltpu.VMEM_SHARED((NPAD, F), jnp.float32),
        pltpu.SemaphoreType.DMA,
    ],
)


@jax.jit
def kernel(x, edge_index, edge_attr, weight):
    w_flat = weight.transpose(2, 0, 1, 3).reshape(F, YW)
    grid = N_NODES // _MM_BLOCK
    y = pl.pallas_call(
        _mm_body,
        grid=(grid,),
        in_specs=[
            pl.BlockSpec((_MM_BLOCK, F), lambda i: (i, 0)),
            pl.BlockSpec((F, YW), lambda i: (0, 0)),
        ],
        out_specs=pl.BlockSpec((_MM_BLOCK, YW), lambda i: (i, 0)),
        out_shape=jax.ShapeDtypeStruct((N_NODES, YW), jnp.float32),
    )(x, w_flat)

    pb = lax.bitcast_convert_type(edge_attr[:, 0], jnp.int32)
    qb = lax.bitcast_convert_type(edge_attr[:, 1], jnp.int32)
    packed = (jnp.stack([pb, qb, edge_index[0], edge_index[1]], axis=0)
              .reshape(4, NCHUNKS, CHUNK).transpose(1, 0, 2))

    partials = _sc_call(y, packed)

    out = pl.pallas_call(
        _add_body,
        grid=(grid,),
        in_specs=[pl.BlockSpec((NC, _MM_BLOCK, F), lambda i: (0, i, 0))],
        out_specs=pl.BlockSpec((_MM_BLOCK, F), lambda i: (i, 0)),
        out_shape=jax.ShapeDtypeStruct((N_NODES, F), jnp.float32),
    )(partials)
    return out
